# Initial kernel scaffold; baseline (speedup 1.0000x reference)
import jax
import jax.numpy as jnp
from jax.experimental import pallas as pl


def kernel(gc_features, cls_id_map, W, b):
    raise NotImplementedError("write your pallas kernel here")


# speedup vs baseline: 1.4284x; 1.4284x over previous
import functools

import jax
import jax.numpy as jnp
from jax import lax
from jax.experimental import pallas as pl
from jax.experimental.pallas import tpu as pltpu
from jax.experimental.pallas import tpu_sc as plsc

_C = 128
_HW = 512 * 512
_K = 65536
_NCLS = 10
_NPAD = 16
_S = 4096


def _classify_block(feat_ref, wp_ref, bp_ref, out_ref):
    f = feat_ref[...]
    logits = lax.dot_general(
        f, wp_ref[...], (((0,), (0,)), ((), ())),
        preferred_element_type=jnp.float32)
    logits = logits + bp_ref[...]
    m = jnp.max(logits, axis=1, keepdims=True)
    e = jnp.exp(logits - m)
    out_ref[...] = e / jnp.sum(e, axis=1, keepdims=True)


def _softmax_table(feat, wp, bp):
    return pl.pallas_call(
        _classify_block,
        grid=(_HW // _S,),
        in_specs=[
            pl.BlockSpec((_C, _S), lambda i: (0, i)),
            pl.BlockSpec((_C, _NPAD), lambda i: (0, 0)),
            pl.BlockSpec((1, _NPAD), lambda i: (0, 0)),
        ],
        out_specs=pl.BlockSpec((_S, _NPAD), lambda i: (i, 0)),
        out_shape=jax.ShapeDtypeStruct((_HW, _NPAD), jnp.float32),
        compiler_params=pltpu.CompilerParams(
            dimension_semantics=("arbitrary",)),
    )(feat, wp, bp)


def _make_row_gather():
    info = plsc.get_sparse_core_info()
    nc, ns = info.num_cores, info.num_subcores
    bpw = _K // (nc * ns)
    mesh = plsc.VectorSubcoreMesh(core_axis_name="c", subcore_axis_name="s")

    @functools.partial(
        pl.kernel, mesh=mesh,
        out_type=jax.ShapeDtypeStruct((_K, _NPAD), jnp.float32),
        scratch_types=[
            pltpu.VMEM((bpw,), jnp.int32),
            pltpu.VMEM((bpw, _NPAD), jnp.float32),
            pltpu.SemaphoreType.DMA,
        ],
        compiler_params=pltpu.CompilerParams(use_tc_tiling_on_sc=False),
    )
    def gather_rows(table_hbm, idx_hbm, out_hbm, idx_v, rows_v, sem):
        wid = lax.axis_index("s") * nc + lax.axis_index("c")
        base = wid * bpw
        pltpu.sync_copy(idx_hbm.at[pl.ds(base, bpw)], idx_v)
        pltpu.async_copy(table_hbm.at[idx_v], rows_v, sem).wait()
        pltpu.sync_copy(rows_v, out_hbm.at[pl.ds(base, bpw)])

    return gather_rows


def kernel(gc_features, cls_id_map, W, b):
    feat = gc_features.reshape(_C, _HW)
    wp = jnp.zeros((_NPAD, _C), jnp.float32).at[:_NCLS, :].set(W).T
    bp = jnp.full((1, _NPAD), -1e30, jnp.float32).at[0, :_NCLS].set(b)
    table = _softmax_table(feat, wp, bp)
    idx = cls_id_map.reshape(_K)
    rows = _make_row_gather()(table, idx)
    return rows[:, :_NCLS]


# speedup vs baseline: 1.8176x; 1.2725x over previous
import functools

import jax
import jax.numpy as jnp
from jax import lax
from jax.experimental import pallas as pl
from jax.experimental.pallas import tpu as pltpu
from jax.experimental.pallas import tpu_sc as plsc

_C = 128
_HW = 512 * 512
_K = 65536
_NCLS = 10
_NPAD = 16
_S = 4096


def _classify_block(feat_ref, wp_ref, bp_ref, out_ref):
    f = feat_ref[...]
    logits = lax.dot_general(
        f, wp_ref[...], (((0,), (0,)), ((), ())),
        preferred_element_type=jnp.float32)
    logits = logits + bp_ref[...]
    m = jnp.max(logits, axis=1, keepdims=True)
    e = jnp.exp(logits - m)
    sm = e / jnp.sum(e, axis=1, keepdims=True)
    out_ref[...] = jnp.concatenate(
        [sm[q * (_S // 8):(q + 1) * (_S // 8), :] for q in range(8)], axis=1)


def _softmax_table(feat, wp, bp):
    return pl.pallas_call(
        _classify_block,
        grid=(_HW // _S,),
        in_specs=[
            pl.BlockSpec((_C, _S), lambda i: (0, i)),
            pl.BlockSpec((_C, _NPAD), lambda i: (0, 0)),
            pl.BlockSpec((1, _NPAD), lambda i: (0, 0)),
        ],
        out_specs=pl.BlockSpec((_S // 8, 128), lambda i: (i, 0)),
        out_shape=jax.ShapeDtypeStruct((_HW // 8, 128), jnp.float32),
        compiler_params=pltpu.CompilerParams(
            dimension_semantics=("arbitrary",)),
    )(feat, wp, bp)


def _make_row_gather():
    info = plsc.get_sparse_core_info()
    nc, ns = info.num_cores, info.num_subcores
    bpw = _K // (nc * ns)
    mesh = plsc.VectorSubcoreMesh(core_axis_name="c", subcore_axis_name="s")

    @functools.partial(
        pl.kernel, mesh=mesh,
        out_type=jax.ShapeDtypeStruct((_K, _NPAD), jnp.float32),
        scratch_types=[
            pltpu.VMEM((bpw,), jnp.int32),
            pltpu.VMEM((bpw,), jnp.int32),
            pltpu.VMEM((bpw, _NPAD), jnp.float32),
            pltpu.SemaphoreType.DMA,
        ],
        compiler_params=pltpu.CompilerParams(use_tc_tiling_on_sc=False),
    )
    def gather_rows(table_hbm, idx_hbm, out_hbm, idx_v, idx2_v, rows_v, sem):
        wid = lax.axis_index("s") * nc + lax.axis_index("c")
        base = wid * bpw
        pltpu.sync_copy(idx_hbm.at[pl.ds(base, bpw)], idx_v)

        def remap(j, carry):
            v = idx_v[pl.ds(j * 16, 16)]
            t = ((v & ~4095) | ((v & 511) << 3)
                 | ((v >> 9) & 7))
            idx2_v[pl.ds(j * 16, 16)] = t
            return carry

        lax.fori_loop(0, bpw // 16, remap, 0)
        pltpu.async_copy(table_hbm.at[idx2_v], rows_v, sem).wait()
        pltpu.sync_copy(rows_v, out_hbm.at[pl.ds(base, bpw)])

    return gather_rows


def kernel(gc_features, cls_id_map, W, b):
    feat = gc_features.reshape(_C, _HW)
    wp = jnp.zeros((_NPAD, _C), jnp.float32).at[:_NCLS, :].set(W).T
    bp = jnp.full((1, _NPAD), -1e30, jnp.float32).at[0, :_NCLS].set(b)
    table = _softmax_table(feat, wp, bp).reshape(_HW, _NPAD)
    idx = cls_id_map.reshape(_K)
    rows = _make_row_gather()(table, idx)
    return rows[:, :_NCLS]
